# baseline (speedup 1.0000x reference)
import jax
import jax.numpy as jnp
import numpy as np
from jax.experimental import pallas as pl
from jax.experimental.pallas import tpu as pltpu

_C = 384
_E = 16
_HW = 784
_G = 48
_GL = 8 * _HW
_INV_TEMP = 2.0

_IND_NP = (np.arange(_GL)[:, None] // _HW == np.arange(8)[None, :])


def _pool_body(x_ref, ind_ref, o_ref):
    xb = x_ref[0].astype(jnp.bfloat16)
    o_ref[0] = jnp.dot(xb, ind_ref[...],
                       preferred_element_type=jnp.float32)


def _gate_body(p_ref, wt_ref, b_ref, o_ref):
    acc = jnp.zeros((p_ref.shape[0], _E), jnp.float32)
    for j in range(8):
        acc += jnp.dot(p_ref[:, :, j], wt_ref[:, j, :],
                       preferred_element_type=jnp.float32)
    logits = (acc * (1.0 / _HW) + b_ref[...]) * _INV_TEMP
    m = jnp.max(logits, axis=-1, keepdims=True)
    e = jnp.exp(logits - m)
    o_ref[...] = e / jnp.sum(e, axis=-1, keepdims=True)


def kernel(x, W, b):
    B = x.shape[0]
    x3 = x.reshape(B, _G, _GL)
    ind = jnp.asarray(_IND_NP, jnp.bfloat16)
    wt3 = W.T.reshape(_G, 8, _E)
    b2 = b.reshape(1, _E)

    praw = pl.pallas_call(
        _pool_body,
        grid=(B,),
        in_specs=[
            pl.BlockSpec((1, _G, _GL), lambda i: (i, 0, 0)),
            pl.BlockSpec((_GL, 8), lambda i: (0, 0)),
        ],
        out_specs=pl.BlockSpec((1, _G, 8), lambda i: (i, 0, 0)),
        out_shape=jax.ShapeDtypeStruct((B, _G, 8), jnp.float32),
    )(x3, ind)

    return pl.pallas_call(
        _gate_body,
        in_specs=[
            pl.BlockSpec((B, _G, 8), lambda: (0, 0, 0)),
            pl.BlockSpec((_G, 8, _E), lambda: (0, 0, 0)),
            pl.BlockSpec((1, _E), lambda: (0, 0)),
        ],
        out_specs=pl.BlockSpec((B, _E), lambda: (0, 0)),
        out_shape=jax.ShapeDtypeStruct((B, _E), jnp.float32),
    )(praw, wt3, b2)


# speedup vs baseline: 15.2180x; 15.2180x over previous
import jax
import jax.numpy as jnp
from jax.experimental import pallas as pl
from jax.experimental.pallas import tpu as pltpu

_E = 16
_INV_TEMP = 2.0
_STEPS = 16


def _router_body(x_ref, wt_ref, b_ref, o_ref, acc_ref):
    i = pl.program_id(0)
    part = jnp.sum(x_ref[...], axis=0)

    @pl.when(i == 0)
    def _init():
        acc_ref[...] = part

    @pl.when(i > 0)
    def _acc():
        acc_ref[...] += part

    @pl.when(i == pl.num_programs(0) - 1)
    def _finish():
        hw = x_ref.shape[0] * pl.num_programs(0)
        pooled = acc_ref[...] * (1.0 / hw)
        logits = jnp.dot(pooled, wt_ref[...],
                         preferred_element_type=jnp.float32)
        logits = (logits + b_ref[...]) * _INV_TEMP
        m = jnp.max(logits, axis=-1, keepdims=True)
        e = jnp.exp(logits - m)
        o_ref[...] = e / jnp.sum(e, axis=-1, keepdims=True)


def kernel(x, W, b):
    B, C = x.shape[0], x.shape[1]
    HW = 1
    for d in x.shape[2:]:
        HW *= d
    xt = jnp.transpose(x, (2, 3, 0, 1)).reshape(HW, B, C)
    wt = W.T
    b2 = b.reshape(1, _E)
    s = HW // _STEPS
    return pl.pallas_call(
        _router_body,
        grid=(_STEPS,),
        in_specs=[
            pl.BlockSpec((s, B, C), lambda i: (i, 0, 0)),
            pl.BlockSpec((C, _E), lambda i: (0, 0)),
            pl.BlockSpec((1, _E), lambda i: (0, 0)),
        ],
        out_specs=pl.BlockSpec((B, _E), lambda i: (0, 0)),
        out_shape=jax.ShapeDtypeStruct((B, _E), jnp.float32),
        scratch_shapes=[pltpu.VMEM((B, C), jnp.float32)],
    )(xt, wt, b2)
